# baseline (speedup 1.0000x reference)
import functools

import jax
import jax.numpy as jnp
from jax import lax
from jax.experimental import pallas as pl
from jax.experimental.pallas import tpu as pltpu
from jax.experimental.pallas import tpu_sc as plsc

N_NODES = 50000
E = 800000
D = 64
NC = 2
NS = 16
L = 16

NB = 51200
SL = NB // NS
EPT = E // NS
CH = 2000
NCHUNK = EPT // CH
NSLOT = 4

_mesh = plsc.VectorSubcoreMesh(
    core_axis_name="c", subcore_axis_name="s", num_cores=NC, num_subcores=NS)
_sc_params = pltpu.CompilerParams(needs_layout_passes=False)


def _unrolled_fori(n, unroll, body):
    assert n % unroll == 0

    def outer(j, _):
        for u in range(unroll):
            body(j * unroll + u)
        return 0

    lax.fori_loop(0, n // unroll, outer, 0)


def _rsqrt16(x):
    i = plsc.bitcast(x, jnp.int32)
    i = 0x5F3759DF - lax.shift_right_logical(i, 1)
    y = plsc.bitcast(i, jnp.float32)
    for _ in range(3):
        y = y * (1.5 - 0.5 * x * y * y)
    return jnp.where(x > 0.0, y, 0.0)


_RSB = 5000


def _rowsums_body(ut_ref, it_ref, au_ref, bu_ref, ai_ref, bi_ref):
    u = ut_ref[...]
    i = it_ref[...]
    au_ref[0, 0, :] = jnp.sum(u[:, :D], axis=1)
    bu_ref[0, 0, :] = jnp.sum(u[:, D:], axis=1)
    ai_ref[0, 0, :] = jnp.sum(i[:, :D], axis=1)
    bi_ref[0, 0, :] = jnp.sum(i[:, D:], axis=1)


_NH = N_NODES // 2
_rowsums = pl.pallas_call(
    _rowsums_body,
    grid=(_NH // _RSB,),
    in_specs=[
        pl.BlockSpec((_RSB, 2 * D), lambda g: (g, 0)),
        pl.BlockSpec((_RSB, 2 * D), lambda g: (g, 0)),
    ],
    out_specs=[
        pl.BlockSpec((1, 1, _RSB), lambda g: (g, 0, 0)),
        pl.BlockSpec((1, 1, _RSB), lambda g: (g, 0, 0)),
        pl.BlockSpec((1, 1, _RSB), lambda g: (g, 0, 0)),
        pl.BlockSpec((1, 1, _RSB), lambda g: (g, 0, 0)),
    ],
    out_shape=[
        jax.ShapeDtypeStruct((_NH // _RSB, 1, _RSB), jnp.float32),
        jax.ShapeDtypeStruct((_NH // _RSB, 1, _RSB), jnp.float32),
        jax.ShapeDtypeStruct((_NH // _RSB, 1, _RSB), jnp.float32),
        jax.ShapeDtypeStruct((_NH // _RSB, 1, _RSB), jnp.float32),
    ],
)


@functools.partial(
    pl.kernel,
    out_type=jax.ShapeDtypeStruct((2 * E,), jnp.float32),
    mesh=_mesh,
    compiler_params=_sc_params,
    scratch_types=[
        pltpu.VMEM((NB,), jnp.float32),
        pltpu.VMEM((NB,), jnp.float32),
        pltpu.VMEM((2 * CH,), jnp.int32),
        pltpu.VMEM((2 * CH,), jnp.int32),
        pltpu.VMEM((2 * CH,), jnp.float32),
        pltpu.VMEM((SL,), jnp.float32),
        pltpu.VMEM_SHARED((NSLOT * NB,), jnp.float32),
        pltpu.SemaphoreType.DMA,
        pltpu.SemaphoreType.DMA,
        pltpu.SemaphoreType.DMA,
        pltpu.SemaphoreType.DMA,
    ],
)
def _fused(tab2, ft, ef_hbm, o2_hbm,
           bufA, bufB, fch, tch, och, gsl, sp,
           sin0, sin1, sout0, sout1):
    c = lax.axis_index("c")
    s = lax.axis_index("s")
    base0 = s * EPT
    zeros16 = jnp.zeros((L,), jnp.float32)
    ones16 = jnp.ones((L,), jnp.float32)

    def zbody(i):
        bufA[pl.ds(i * L, L)] = zeros16

    _unrolled_fori(NB // L, 8, zbody)

    def _h_start(k, buf, sem):
        pltpu.make_async_copy(
            ef_hbm.at[pl.ds(E + base0 + k * CH, CH)],
            fch.at[pl.ds(buf * CH, CH)], sem).start()

    def _h_wait(k, buf, sem):
        pltpu.make_async_copy(
            ef_hbm.at[pl.ds(E + base0 + k * CH, CH)],
            fch.at[pl.ds(buf * CH, CH)], sem).wait()

    def _h_compute(buf):
        def body(i):
            plsc.addupdate_scatter(
                bufA, [fch[pl.ds(buf * CH + i * L, L)]], ones16)

        _unrolled_fori(CH // L, 5, body)

    _h_start(0, 0, sin0)
    _h_start(1, 1, sin1)

    def h_pair(j, _):
        k0 = j * 2
        _h_wait(k0, 0, sin0)
        _h_compute(0)

        @pl.when(k0 + 2 < NCHUNK)
        def _():
            _h_start(k0 + 2, 0, sin0)

        _h_wait(k0 + 1, 1, sin1)
        _h_compute(1)

        @pl.when(k0 + 3 < NCHUNK)
        def _():
            _h_start(k0 + 3, 1, sin1)

        return 0

    lax.fori_loop(0, (NCHUNK - 1) // 2, h_pair, 0)
    _h_wait(NCHUNK - 1, 0, sin0)
    _h_compute(0)

    for r in range(NS // NSLOT):
        @pl.when((s >= NSLOT * r) & (s < NSLOT * (r + 1)))
        def _():
            pltpu.sync_copy(bufA, sp.at[pl.ds((s - NSLOT * r) * NB, NB)])

        plsc.subcore_barrier()
        for j in range(NSLOT):
            pltpu.sync_copy(sp.at[pl.ds(j * NB + s * SL, SL)], gsl)
            first = (r == 0 and j == 0)

            def abody(i, first=first):
                sl = pl.ds(i * L, L)
                if first:
                    och[sl] = gsl[sl]
                else:
                    och[sl] = och[sl] + gsl[sl]

            _unrolled_fori(SL // L, 8, abody)
        plsc.subcore_barrier()

    def dbody(i):
        sl = pl.ds(i * L, L)
        och[sl] = _rsqrt16(och[sl])

    _unrolled_fori(SL // L, 4, dbody)

    pltpu.sync_copy(tab2.at[pl.ds(c * N_NODES, N_NODES)],
                    bufB.at[pl.ds(0, N_NODES)])
    pltpu.sync_copy(ft.at[pl.ds(c * NB + s * SL, SL)], fch.at[pl.ds(0, SL)])

    def gbody(i):
        sl = pl.ds(i * L, L)
        n = fch[sl]
        idx = (n & 1) * (N_NODES // 2) + lax.shift_right_logical(n, 1)
        gsl[sl] = och[sl] * plsc.load_gather(bufB, [idx])

    _unrolled_fori(SL // L, 5, gbody)
    pltpu.sync_copy(och.at[pl.ds(0, SL)], sp.at[pl.ds(s * SL, SL)])
    pltpu.sync_copy(gsl, sp.at[pl.ds(NB + s * SL, SL)])
    plsc.subcore_barrier()

    pltpu.sync_copy(sp.at[pl.ds(0, NB)], bufA)
    pltpu.sync_copy(sp.at[pl.ds(NB, NB)], bufB)
    obase0 = c * E + base0

    def _start_in(k, buf, sem):
        pltpu.make_async_copy(
            ef_hbm.at[pl.ds(base0 + k * CH, CH)],
            fch.at[pl.ds(buf * CH, CH)], sem).start()
        pltpu.make_async_copy(
            ef_hbm.at[pl.ds(E + base0 + k * CH, CH)],
            tch.at[pl.ds(buf * CH, CH)], sem).start()

    def _wait_in(k, buf, sem):
        pltpu.make_async_copy(
            ef_hbm.at[pl.ds(base0 + k * CH, CH)],
            fch.at[pl.ds(buf * CH, CH)], sem).wait()
        pltpu.make_async_copy(
            ef_hbm.at[pl.ds(E + base0 + k * CH, CH)],
            tch.at[pl.ds(buf * CH, CH)], sem).wait()

    def _start_out(k, buf, sem):
        pltpu.make_async_copy(
            och.at[pl.ds(buf * CH, CH)],
            o2_hbm.at[pl.ds(obase0 + k * CH, CH)], sem).start()

    def _wait_out(k, buf, sem):
        pltpu.make_async_copy(
            och.at[pl.ds(buf * CH, CH)],
            o2_hbm.at[pl.ds(obase0 + k * CH, CH)], sem).wait()

    def _compute(buf):
        def body(i):
            sl = pl.ds(buf * CH + i * L, L)
            och[sl] = (plsc.load_gather(bufB, [fch[sl]])
                       * plsc.load_gather(bufA, [tch[sl]]))

        _unrolled_fori(CH // L, 5, body)

    _start_in(0, 0, sin0)
    _start_in(1, 1, sin1)

    def pair(j, _):
        k0 = j * 2

        @pl.when(j > 0)
        def _():
            _wait_out(k0 - 2, 0, sout0)

        _wait_in(k0, 0, sin0)
        _compute(0)
        _start_out(k0, 0, sout0)

        @pl.when(k0 + 2 < NCHUNK)
        def _():
            _start_in(k0 + 2, 0, sin0)

        @pl.when(j > 0)
        def _():
            _wait_out(k0 - 1, 1, sout1)

        _wait_in(k0 + 1, 1, sin1)
        _compute(1)
        _start_out(k0 + 1, 1, sout1)

        @pl.when(k0 + 3 < NCHUNK)
        def _():
            _start_in(k0 + 3, 1, sin1)

        return 0

    lax.fori_loop(0, (NCHUNK - 1) // 2, pair, 0)
    k_last = NCHUNK - 1
    _wait_out(k_last - 2, 0, sout0)
    _wait_out(k_last - 1, 1, sout1)
    _wait_in(k_last, 0, sin0)
    _compute(0)
    _start_out(k_last, 0, sout0)
    _wait_out(k_last, 0, sout0)


@jax.jit
def kernel(edge_index, user_table, item_table):
    ef = edge_index.reshape(2 * E)

    au, bu, ai, bi = _rowsums(user_table.reshape(_NH, 2 * D),
                              item_table.reshape(_NH, 2 * D))
    tab2 = jnp.concatenate([au.reshape(_NH), bu.reshape(_NH),
                            ai.reshape(_NH), bi.reshape(_NH)])
    ft = jnp.concatenate([ef[:NB], ef[E:E + NB]])
    o2 = _fused(tab2, ft, ef)
    return (o2[:E], o2[E:])


# speedup vs baseline: 1.0588x; 1.0588x over previous
import functools

import jax
import jax.numpy as jnp
from jax import lax
from jax.experimental import pallas as pl
from jax.experimental.pallas import tpu as pltpu
from jax.experimental.pallas import tpu_sc as plsc

N_NODES = 50000
E = 800000
D = 64
NC = 2
NS = 16
L = 16

NB = 51200
SL = NB // NS
EPT = E // NS
CH = 2000
NCHUNK = EPT // CH
NSLOT = 4

_mesh = plsc.VectorSubcoreMesh(
    core_axis_name="c", subcore_axis_name="s", num_cores=NC, num_subcores=NS)
_sc_params = pltpu.CompilerParams(needs_layout_passes=False)


def _unrolled_fori(n, unroll, body):
    assert n % unroll == 0

    def outer(j, _):
        for u in range(unroll):
            body(j * unroll + u)
        return 0

    lax.fori_loop(0, n // unroll, outer, 0)


def _rsqrt16(x):
    i = plsc.bitcast(x, jnp.int32)
    i = 0x5F3759DF - lax.shift_right_logical(i, 1)
    y = plsc.bitcast(i, jnp.float32)
    for _ in range(3):
        y = y * (1.5 - 0.5 * x * y * y)
    return jnp.where(x > 0.0, y, 0.0)


_RSB = 10000


def _rowsums_body(ut_ref, it_ref, su_ref, si_ref):
    ones_d = jnp.ones((D,), jnp.float32)
    su_ref[0, 0, :] = jnp.matmul(ut_ref[...], ones_d)
    si_ref[0, 0, :] = jnp.matmul(it_ref[...], ones_d)


_rowsums = pl.pallas_call(
    _rowsums_body,
    grid=(N_NODES // _RSB,),
    in_specs=[
        pl.BlockSpec((_RSB, D), lambda g: (g, 0)),
        pl.BlockSpec((_RSB, D), lambda g: (g, 0)),
    ],
    out_specs=[
        pl.BlockSpec((1, 1, _RSB), lambda g: (g, 0, 0)),
        pl.BlockSpec((1, 1, _RSB), lambda g: (g, 0, 0)),
    ],
    out_shape=[
        jax.ShapeDtypeStruct((N_NODES // _RSB, 1, _RSB), jnp.float32),
        jax.ShapeDtypeStruct((N_NODES // _RSB, 1, _RSB), jnp.float32),
    ],
)


@functools.partial(
    pl.kernel,
    out_type=jax.ShapeDtypeStruct((2 * E,), jnp.float32),
    mesh=_mesh,
    compiler_params=_sc_params,
    scratch_types=[
        pltpu.VMEM((NB,), jnp.float32),
        pltpu.VMEM((NB,), jnp.float32),
        pltpu.VMEM((2 * CH,), jnp.int32),
        pltpu.VMEM((2 * CH,), jnp.int32),
        pltpu.VMEM((2 * CH,), jnp.float32),
        pltpu.VMEM((SL,), jnp.float32),
        pltpu.VMEM_SHARED((NSLOT * NB,), jnp.float32),
        pltpu.SemaphoreType.DMA,
        pltpu.SemaphoreType.DMA,
        pltpu.SemaphoreType.DMA,
        pltpu.SemaphoreType.DMA,
    ],
)
def _fused(tab2, ft, ef_hbm, o2_hbm,
           bufA, bufB, fch, tch, och, gsl, sp,
           sin0, sin1, sout0, sout1):
    c = lax.axis_index("c")
    s = lax.axis_index("s")
    base0 = s * EPT
    zeros16 = jnp.zeros((L,), jnp.float32)
    ones16 = jnp.ones((L,), jnp.float32)

    def zbody(i):
        bufA[pl.ds(i * L, L)] = zeros16

    _unrolled_fori(NB // L, 8, zbody)

    def _h_start(k, buf, sem):
        pltpu.make_async_copy(
            ef_hbm.at[pl.ds(E + base0 + k * CH, CH)],
            fch.at[pl.ds(buf * CH, CH)], sem).start()

    def _h_wait(k, buf, sem):
        pltpu.make_async_copy(
            ef_hbm.at[pl.ds(E + base0 + k * CH, CH)],
            fch.at[pl.ds(buf * CH, CH)], sem).wait()

    def _h_compute(buf):
        def body(i):
            plsc.addupdate_scatter(
                bufA, [fch[pl.ds(buf * CH + i * L, L)]], ones16)

        _unrolled_fori(CH // L, 25, body)

    _h_start(0, 0, sin0)
    _h_start(1, 1, sin1)

    def h_pair(j, _):
        k0 = j * 2
        _h_wait(k0, 0, sin0)
        _h_compute(0)

        @pl.when(k0 + 2 < NCHUNK)
        def _():
            _h_start(k0 + 2, 0, sin0)

        _h_wait(k0 + 1, 1, sin1)
        _h_compute(1)

        @pl.when(k0 + 3 < NCHUNK)
        def _():
            _h_start(k0 + 3, 1, sin1)

        return 0

    lax.fori_loop(0, (NCHUNK - 1) // 2, h_pair, 0)
    _h_wait(NCHUNK - 1, 0, sin0)
    _h_compute(0)

    for r in range(NS // NSLOT):
        @pl.when((s >= NSLOT * r) & (s < NSLOT * (r + 1)))
        def _():
            pltpu.sync_copy(bufA, sp.at[pl.ds((s - NSLOT * r) * NB, NB)])

        plsc.subcore_barrier()
        for j in range(NSLOT):
            pltpu.sync_copy(sp.at[pl.ds(j * NB + s * SL, SL)], gsl)
            first = (r == 0 and j == 0)

            def abody(i, first=first):
                sl = pl.ds(i * L, L)
                if first:
                    och[sl] = gsl[sl]
                else:
                    och[sl] = och[sl] + gsl[sl]

            _unrolled_fori(SL // L, 8, abody)
        plsc.subcore_barrier()

    def dbody(i):
        sl = pl.ds(i * L, L)
        och[sl] = _rsqrt16(och[sl])

    _unrolled_fori(SL // L, 4, dbody)

    pltpu.sync_copy(tab2.at[pl.ds(c * N_NODES, N_NODES)],
                    bufB.at[pl.ds(0, N_NODES)])
    pltpu.sync_copy(ft.at[pl.ds(c * NB + s * SL, SL)], fch.at[pl.ds(0, SL)])

    def gbody(i):
        sl = pl.ds(i * L, L)
        gsl[sl] = och[sl] * plsc.load_gather(bufB, [fch[sl]])

    _unrolled_fori(SL // L, 25, gbody)
    pltpu.sync_copy(och.at[pl.ds(0, SL)], sp.at[pl.ds(s * SL, SL)])
    pltpu.sync_copy(gsl, sp.at[pl.ds(NB + s * SL, SL)])
    plsc.subcore_barrier()

    pltpu.sync_copy(sp.at[pl.ds(0, NB)], bufA)
    pltpu.sync_copy(sp.at[pl.ds(NB, NB)], bufB)
    obase0 = c * E + base0

    def _start_in(k, buf, sem):
        pltpu.make_async_copy(
            ef_hbm.at[pl.ds(base0 + k * CH, CH)],
            fch.at[pl.ds(buf * CH, CH)], sem).start()
        pltpu.make_async_copy(
            ef_hbm.at[pl.ds(E + base0 + k * CH, CH)],
            tch.at[pl.ds(buf * CH, CH)], sem).start()

    def _wait_in(k, buf, sem):
        pltpu.make_async_copy(
            ef_hbm.at[pl.ds(base0 + k * CH, CH)],
            fch.at[pl.ds(buf * CH, CH)], sem).wait()
        pltpu.make_async_copy(
            ef_hbm.at[pl.ds(E + base0 + k * CH, CH)],
            tch.at[pl.ds(buf * CH, CH)], sem).wait()

    def _start_out(k, buf, sem):
        pltpu.make_async_copy(
            och.at[pl.ds(buf * CH, CH)],
            o2_hbm.at[pl.ds(obase0 + k * CH, CH)], sem).start()

    def _wait_out(k, buf, sem):
        pltpu.make_async_copy(
            och.at[pl.ds(buf * CH, CH)],
            o2_hbm.at[pl.ds(obase0 + k * CH, CH)], sem).wait()

    def _compute(buf):
        def body(i):
            sl = pl.ds(buf * CH + i * L, L)
            och[sl] = (plsc.load_gather(bufB, [fch[sl]])
                       * plsc.load_gather(bufA, [tch[sl]]))

        _unrolled_fori(CH // L, 25, body)

    _start_in(0, 0, sin0)
    _start_in(1, 1, sin1)

    def pair(j, _):
        k0 = j * 2

        @pl.when(j > 0)
        def _():
            _wait_out(k0 - 2, 0, sout0)

        _wait_in(k0, 0, sin0)
        _compute(0)
        _start_out(k0, 0, sout0)

        @pl.when(k0 + 2 < NCHUNK)
        def _():
            _start_in(k0 + 2, 0, sin0)

        @pl.when(j > 0)
        def _():
            _wait_out(k0 - 1, 1, sout1)

        _wait_in(k0 + 1, 1, sin1)
        _compute(1)
        _start_out(k0 + 1, 1, sout1)

        @pl.when(k0 + 3 < NCHUNK)
        def _():
            _start_in(k0 + 3, 1, sin1)

        return 0

    lax.fori_loop(0, (NCHUNK - 1) // 2, pair, 0)
    k_last = NCHUNK - 1
    _wait_out(k_last - 2, 0, sout0)
    _wait_out(k_last - 1, 1, sout1)
    _wait_in(k_last, 0, sin0)
    _compute(0)
    _start_out(k_last, 0, sout0)
    _wait_out(k_last, 0, sout0)


@jax.jit
def kernel(edge_index, user_table, item_table):
    ef = edge_index.reshape(2 * E)

    su3, si3 = _rowsums(user_table, item_table)
    tab2 = jnp.concatenate([su3.reshape(N_NODES), si3.reshape(N_NODES)])
    ft = jnp.concatenate([ef[:NB], ef[E:E + NB]])
    o2 = _fused(tab2, ft, ef)
    return (o2[:E], o2[E:])


# speedup vs baseline: 1.0953x; 1.0344x over previous
import functools

import jax
import jax.numpy as jnp
from jax import lax
from jax.experimental import pallas as pl
from jax.experimental.pallas import tpu as pltpu
from jax.experimental.pallas import tpu_sc as plsc

N_NODES = 50000
E = 800000
D = 64
NC = 2
NS = 16
L = 16

NB = 51200
SL = NB // NS
EPT = E // NS
CH = 2000
NCHUNK = EPT // CH
NSLOT = 4

_mesh = plsc.VectorSubcoreMesh(
    core_axis_name="c", subcore_axis_name="s", num_cores=NC, num_subcores=NS)
_sc_params = pltpu.CompilerParams(needs_layout_passes=False)


def _unrolled_fori(n, unroll, body):
    assert n % unroll == 0

    def outer(j, _):
        for u in range(unroll):
            body(j * unroll + u)
        return 0

    lax.fori_loop(0, n // unroll, outer, 0)


def _rsqrt16(x):
    i = plsc.bitcast(x, jnp.int32)
    i = 0x5F3759DF - lax.shift_right_logical(i, 1)
    y = plsc.bitcast(i, jnp.float32)
    for _ in range(3):
        y = y * (1.5 - 0.5 * x * y * y)
    return jnp.where(x > 0.0, y, 0.0)


_RSB = 10000


def _rowsums_body(ut_ref, it_ref, su_ref, si_ref):
    ones_d = jnp.ones((D,), jnp.float32)
    su_ref[0, 0, :] = jnp.matmul(ut_ref[...], ones_d)
    si_ref[0, 0, :] = jnp.matmul(it_ref[...], ones_d)


_rowsums = pl.pallas_call(
    _rowsums_body,
    grid=(N_NODES // _RSB,),
    in_specs=[
        pl.BlockSpec((_RSB, D), lambda g: (g, 0)),
        pl.BlockSpec((_RSB, D), lambda g: (g, 0)),
    ],
    out_specs=[
        pl.BlockSpec((1, 1, _RSB), lambda g: (g, 0, 0)),
        pl.BlockSpec((1, 1, _RSB), lambda g: (g, 0, 0)),
    ],
    out_shape=[
        jax.ShapeDtypeStruct((N_NODES // _RSB, 1, _RSB), jnp.float32),
        jax.ShapeDtypeStruct((N_NODES // _RSB, 1, _RSB), jnp.float32),
    ],
)


@functools.partial(
    pl.kernel,
    out_type=jax.ShapeDtypeStruct((2 * E,), jnp.float32),
    mesh=_mesh,
    compiler_params=_sc_params,
    scratch_types=[
        pltpu.VMEM((NB,), jnp.float32),
        pltpu.VMEM((NB,), jnp.float32),
        pltpu.VMEM((2 * CH,), jnp.int32),
        pltpu.VMEM((2 * CH,), jnp.int32),
        pltpu.VMEM((2 * CH,), jnp.float32),
        pltpu.VMEM((SL,), jnp.float32),
        pltpu.VMEM_SHARED((NSLOT * NB,), jnp.float32),
        pltpu.SemaphoreType.DMA,
        pltpu.SemaphoreType.DMA,
        pltpu.SemaphoreType.DMA,
        pltpu.SemaphoreType.DMA,
    ],
)
def _fused(tab2, ft, ef_hbm, o2_hbm,
           bufA, bufB, fch, tch, och, gsl, sp,
           sin0, sin1, sout0, sout1):
    c = lax.axis_index("c")
    s = lax.axis_index("s")
    base0 = s * EPT
    zeros16 = jnp.zeros((L,), jnp.float32)
    ones16 = jnp.ones((L,), jnp.float32)

    def zbody(i):
        bufA[pl.ds(i * L, L)] = zeros16

    _unrolled_fori(NB // L, 8, zbody)

    def _h_start(k, buf, sem):
        pltpu.make_async_copy(
            ef_hbm.at[pl.ds(E + base0 + k * CH, CH)],
            fch.at[pl.ds(buf * CH, CH)], sem).start()

    def _h_wait(k, buf, sem):
        pltpu.make_async_copy(
            ef_hbm.at[pl.ds(E + base0 + k * CH, CH)],
            fch.at[pl.ds(buf * CH, CH)], sem).wait()

    def _h_compute(buf):
        def body(i):
            plsc.addupdate_scatter(
                bufA, [fch[pl.ds(buf * CH + i * L, L)]], ones16)

        _unrolled_fori(CH // L, 5, body)

    _h_start(0, 0, sin0)
    _h_start(1, 1, sin1)

    def h_pair(j, _):
        k0 = j * 2
        _h_wait(k0, 0, sin0)
        _h_compute(0)

        @pl.when(k0 + 2 < NCHUNK)
        def _():
            _h_start(k0 + 2, 0, sin0)

        _h_wait(k0 + 1, 1, sin1)
        _h_compute(1)

        @pl.when(k0 + 3 < NCHUNK)
        def _():
            _h_start(k0 + 3, 1, sin1)

        return 0

    lax.fori_loop(0, (NCHUNK - 1) // 2, h_pair, 0)
    _h_wait(NCHUNK - 1, 0, sin0)
    _h_compute(0)

    for r in range(NS // NSLOT):
        @pl.when((s >= NSLOT * r) & (s < NSLOT * (r + 1)))
        def _():
            pltpu.sync_copy(bufA, sp.at[pl.ds((s - NSLOT * r) * NB, NB)])

        plsc.subcore_barrier()
        for j in range(NSLOT):
            pltpu.sync_copy(sp.at[pl.ds(j * NB + s * SL, SL)], gsl)
            first = (r == 0 and j == 0)

            def abody(i, first=first):
                sl = pl.ds(i * L, L)
                if first:
                    och[sl] = gsl[sl]
                else:
                    och[sl] = och[sl] + gsl[sl]

            _unrolled_fori(SL // L, 8, abody)
        plsc.subcore_barrier()

    def dbody(i):
        sl = pl.ds(i * L, L)
        och[sl] = _rsqrt16(och[sl])

    _unrolled_fori(SL // L, 4, dbody)

    pltpu.sync_copy(tab2.at[pl.ds(c * N_NODES, N_NODES)],
                    bufB.at[pl.ds(0, N_NODES)])
    pltpu.sync_copy(ft.at[pl.ds(c * NB + s * SL, SL)], fch.at[pl.ds(0, SL)])

    def gbody(i):
        sl = pl.ds(i * L, L)
        gsl[sl] = och[sl] * plsc.load_gather(bufB, [fch[sl]])

    _unrolled_fori(SL // L, 5, gbody)
    pltpu.sync_copy(och.at[pl.ds(0, SL)], sp.at[pl.ds(s * SL, SL)])
    pltpu.sync_copy(gsl, sp.at[pl.ds(NB + s * SL, SL)])
    plsc.subcore_barrier()

    pltpu.sync_copy(sp.at[pl.ds(0, NB)], bufA)
    pltpu.sync_copy(sp.at[pl.ds(NB, NB)], bufB)
    obase0 = c * E + base0

    def _start_in(k, buf, sem):
        pltpu.make_async_copy(
            ef_hbm.at[pl.ds(base0 + k * CH, CH)],
            fch.at[pl.ds(buf * CH, CH)], sem).start()
        pltpu.make_async_copy(
            ef_hbm.at[pl.ds(E + base0 + k * CH, CH)],
            tch.at[pl.ds(buf * CH, CH)], sem).start()

    def _wait_in(k, buf, sem):
        pltpu.make_async_copy(
            ef_hbm.at[pl.ds(base0 + k * CH, CH)],
            fch.at[pl.ds(buf * CH, CH)], sem).wait()
        pltpu.make_async_copy(
            ef_hbm.at[pl.ds(E + base0 + k * CH, CH)],
            tch.at[pl.ds(buf * CH, CH)], sem).wait()

    def _start_out(k, buf, sem):
        pltpu.make_async_copy(
            och.at[pl.ds(buf * CH, CH)],
            o2_hbm.at[pl.ds(obase0 + k * CH, CH)], sem).start()

    def _wait_out(k, buf, sem):
        pltpu.make_async_copy(
            och.at[pl.ds(buf * CH, CH)],
            o2_hbm.at[pl.ds(obase0 + k * CH, CH)], sem).wait()

    def _compute(buf):
        def body(i):
            sl = pl.ds(buf * CH + i * L, L)
            och[sl] = (plsc.load_gather(bufB, [fch[sl]])
                       * plsc.load_gather(bufA, [tch[sl]]))

        _unrolled_fori(CH // L, 5, body)

    _start_in(0, 0, sin0)
    _start_in(1, 1, sin1)

    def pair(j, _):
        k0 = j * 2

        @pl.when(j > 0)
        def _():
            _wait_out(k0 - 2, 0, sout0)

        _wait_in(k0, 0, sin0)
        _compute(0)
        _start_out(k0, 0, sout0)

        @pl.when(k0 + 2 < NCHUNK)
        def _():
            _start_in(k0 + 2, 0, sin0)

        @pl.when(j > 0)
        def _():
            _wait_out(k0 - 1, 1, sout1)

        _wait_in(k0 + 1, 1, sin1)
        _compute(1)
        _start_out(k0 + 1, 1, sout1)

        @pl.when(k0 + 3 < NCHUNK)
        def _():
            _start_in(k0 + 3, 1, sin1)

        return 0

    lax.fori_loop(0, (NCHUNK - 1) // 2, pair, 0)
    k_last = NCHUNK - 1
    _wait_out(k_last - 2, 0, sout0)
    _wait_out(k_last - 1, 1, sout1)
    _wait_in(k_last, 0, sin0)
    _compute(0)
    _start_out(k_last, 0, sout0)
    _wait_out(k_last, 0, sout0)


@jax.jit
def kernel(edge_index, user_table, item_table):
    ef = edge_index.reshape(2 * E)

    su3, si3 = _rowsums(user_table, item_table)
    tab2 = jnp.concatenate([su3.reshape(N_NODES), si3.reshape(N_NODES)])
    ft = jnp.concatenate([ef[:NB], ef[E:E + NB]])
    o2 = _fused(tab2, ft, ef)
    return (o2[:E], o2[E:])
